# baseline (speedup 1.0000x reference)
import functools

import jax
import jax.numpy as jnp
from jax import lax
from jax.experimental import pallas as pl
from jax.experimental.pallas import tpu as pltpu
from jax.experimental.pallas import tpu_sc as plsc

VOCAB = 1000
D = 1000
B = 4096 * 20

NC, NS = 2, 16
NW = NC * NS
C = 64
B_PER_W = B // NW
N_CHUNKS = B_PER_W // C


def _body(idx_hbm, tab_hbm, out_hbm, idx_v, buf0, buf1, g0, g1, s0, s1):
    wid = lax.axis_index("s") * NC + lax.axis_index("c")
    base = wid * B_PER_W
    pltpu.sync_copy(idx_hbm.at[wid], idx_v)

    def gather(j, buf, sem):
        return pltpu.make_async_copy(tab_hbm.at[idx_v.at[j]], buf, sem)

    def store(j, buf, sem):
        return pltpu.make_async_copy(buf, out_hbm.at[pl.ds(base + j * C, C)], sem)

    gather(0, buf0, g0).start()

    def pair(i, carry):
        j = 2 * i
        gather(j, buf0, g0).wait()
        store(j, buf0, s0).start()

        @pl.when(j >= 2)
        def _():
            store(j - 1, buf1, s1).wait()

        gather(j + 1, buf1, g1).start()
        gather(j + 1, buf1, g1).wait()
        store(j + 1, buf1, s1).start()

        @pl.when(j + 2 < N_CHUNKS)
        def _():
            store(j, buf0, s0).wait()
            gather(j + 2, buf0, g0).start()

        return carry

    lax.fori_loop(0, N_CHUNKS // 2, pair, 0)
    store(N_CHUNKS - 2, buf0, s0).wait()
    store(N_CHUNKS - 1, buf1, s1).wait()


@functools.partial(jax.jit, static_argnums=())
def _gather_rows(idx, table):
    k = pl.kernel(
        _body,
        out_type=jax.ShapeDtypeStruct((B, D), jnp.float32),
        mesh=plsc.VectorSubcoreMesh(core_axis_name="c", subcore_axis_name="s"),
        scratch_types=[
            pltpu.VMEM((N_CHUNKS, C), jnp.int32),
            pltpu.VMEM((C, D), jnp.float32),
            pltpu.VMEM((C, D), jnp.float32),
            pltpu.SemaphoreType.DMA,
            pltpu.SemaphoreType.DMA,
            pltpu.SemaphoreType.DMA,
            pltpu.SemaphoreType.DMA,
        ],
        compiler_params=pltpu.CompilerParams(use_tc_tiling_on_sc=False),
    )
    return k(idx, table)


def kernel(inputs, embedding_table):
    idx = inputs.reshape(NW, N_CHUNKS, C)
    out = _gather_rows(idx, embedding_table)
    return out.reshape(inputs.shape + (embedding_table.shape[1],))
